# Initial kernel scaffold; baseline (speedup 1.0000x reference)
import jax
import jax.numpy as jnp
from jax.experimental import pallas as pl


def kernel(x, edge_index, W_self, W_neigh, b):
    raise NotImplementedError("write your pallas kernel here")


# speedup vs baseline: 1.5811x; 1.5811x over previous
import functools

import jax
import jax.numpy as jnp
from jax import lax
from jax.experimental import pallas as pl
from jax.experimental.pallas import tpu as pltpu
from jax.experimental.pallas import tpu_sc as plsc

N_NODES = 10000
N_EDGES = 160000
D = 256

NC = 2
NS = 16
NW = NC * NS
LANES = 16

PART = 320
NPAD = NW * PART
ACC_ROWS = 336
TRASH = PART
CH = 64
CAP = 128
SCH = 2000
NSCH = N_EDGES // SCH
DEGC = PART // LANES


def _sc_gather_scatter(x, src_ids, dst_ids):
  mesh = plsc.VectorSubcoreMesh(core_axis_name="c", subcore_axis_name="s")

  @functools.partial(
      pl.kernel,
      out_type=[
          jax.ShapeDtypeStruct((NPAD, D), jnp.float32),
          jax.ShapeDtypeStruct((NW, DEGC, LANES), jnp.float32),
      ],
      mesh=mesh,
      compiler_params=pltpu.CompilerParams(needs_layout_passes=False),
      scratch_types=[
          pltpu.VMEM((SCH,), jnp.int32),
          pltpu.VMEM((SCH,), jnp.int32),
          pltpu.VMEM((CAP,), jnp.int32),
          pltpu.VMEM((CAP,), jnp.int32),
          pltpu.VMEM((CH, D), jnp.float32),
          pltpu.VMEM((ACC_ROWS, D), jnp.float32),
          pltpu.VMEM((ACC_ROWS + LANES,), jnp.float32),
          pltpu.VMEM((DEGC, LANES), jnp.float32),
          pltpu.SemaphoreType.DMA,
      ],
  )
  def k(x_hbm, si_hbm, di_hbm, a_out, deg_out, src_v, dst_v, psrc, pdst,
        gbuf, acc, deg_v, deg2, sem):
    c = lax.axis_index("c")
    s = lax.axis_index("s")
    wid = s * NC + c
    lo = wid * PART

    zvec = jnp.zeros((LANES,), jnp.float32)
    onevec = jnp.where(lax.iota(jnp.int32, LANES) == 0, 1.0, 0.0)

    def zero_acc(r, _):
      for v in range(D // LANES):
        acc[r, pl.ds(v * LANES, LANES)] = zvec
      return 0
    lax.fori_loop(0, ACC_ROWS, zero_acc, 0)

    def zero_deg(r, _):
      deg_v[pl.ds(r * LANES, LANES)] = zvec
      return 0
    lax.fori_loop(0, (ACC_ROWS + LANES) // LANES, zero_deg, 0)

    def drain():
      pltpu.async_copy(x_hbm.at[psrc.at[pl.ds(0, CH)]], gbuf, sem).wait()

      def group(g, _):
        rows = pdst[pl.ds(g * LANES, LANES)]
        for lane in range(LANES):
          row = rows[lane]
          e = g * LANES + lane
          for v in range(D // LANES):
            sl = pl.ds(v * LANES, LANES)
            acc[row, sl] = acc[row, sl] + gbuf[e, sl]
          dw = deg_v[pl.ds(row, LANES)]
          deg_v[pl.ds(row, LANES)] = dw + onevec
        return 0
      lax.fori_loop(0, CH // LANES, group, 0)

    def superstep(g, cursor):
      pltpu.sync_copy(si_hbm.at[pl.ds(g * SCH, SCH)], src_v)
      pltpu.sync_copy(di_hbm.at[pl.ds(g * SCH, SCH)], dst_v)

      def scan(r, cur):
        dvec = dst_v[pl.ds(r * LANES, LANES)]
        svec = src_v[pl.ds(r * LANES, LANES)]
        mine = (dvec >= lo) & (dvec < lo + PART)
        mcum = plsc.cumsum(jnp.where(mine, jnp.int32(1), jnp.int32(0)))
        idx = cur + mcum - 1
        plsc.store_scatter(psrc, [idx], svec, mask=mine)
        plsc.store_scatter(pdst, [idx], dvec - lo, mask=mine)
        cur = cur + mcum[LANES - 1]

        def do_drain(cc):
          drain()
          psrc[pl.ds(0, LANES)] = psrc[pl.ds(CH, LANES)]
          pdst[pl.ds(0, LANES)] = pdst[pl.ds(CH, LANES)]
          return cc - CH
        return lax.cond(cur >= CH, do_drain, lambda cc: cc, cur)
      return lax.fori_loop(0, SCH // LANES, scan, cursor)
    cursor = lax.fori_loop(0, NSCH, superstep, jnp.int32(0))

    zivec = jnp.zeros((LANES,), jnp.int32)
    tvec = jnp.full((LANES,), TRASH, jnp.int32)
    def pad(w, _):
      off = cursor + w * LANES
      psrc[pl.ds(off, LANES)] = zivec
      pdst[pl.ds(off, LANES)] = tvec
      return 0
    lax.fori_loop(0, CH // LANES, pad, 0)
    drain()

    pltpu.sync_copy(acc.at[pl.ds(0, PART)], a_out.at[pl.ds(lo, PART)])
    def stage_deg(r, _):
      deg2[r, :] = deg_v[pl.ds(r * LANES, LANES)]
      return 0
    lax.fori_loop(0, DEGC, stage_deg, 0)
    pltpu.sync_copy(deg2, deg_out.at[wid])

  return k(x, src_ids, dst_ids)


def _tc_dense(x, a, deg, w_self, w_neigh, b2):
  BLK = 1000
  grid = N_NODES // BLK
  deg3 = deg.reshape(grid, 1, BLK)

  def body(x_ref, a_ref, deg_ref, ws_ref, wn_ref, b_ref, o_ref):
    inv = 1.0 / jnp.maximum(deg_ref[0, 0, :], 1.0)
    acc = jnp.dot(x_ref[...], ws_ref[...], preferred_element_type=jnp.float32)
    agg = jnp.dot(a_ref[...], wn_ref[...], preferred_element_type=jnp.float32)
    o_ref[...] = jnp.maximum(acc + agg * inv[:, None] + b_ref[...], 0.0)

  return pl.pallas_call(
      body,
      grid=(grid,),
      in_specs=[
          pl.BlockSpec((BLK, D), lambda i: (i, 0)),
          pl.BlockSpec((BLK, D), lambda i: (i, 0)),
          pl.BlockSpec((1, 1, BLK), lambda i: (i, 0, 0)),
          pl.BlockSpec((D, D), lambda i: (0, 0)),
          pl.BlockSpec((D, D), lambda i: (0, 0)),
          pl.BlockSpec((1, D), lambda i: (0, 0)),
      ],
      out_specs=pl.BlockSpec((BLK, D), lambda i: (i, 0)),
      out_shape=jax.ShapeDtypeStruct((N_NODES, D), jnp.float32),
  )(x, a, deg3, w_self, w_neigh, b2)


def kernel(x, edge_index, W_self, W_neigh, b):
  a_pad, deg_raw = _sc_gather_scatter(x, edge_index[0], edge_index[1])
  a = a_pad[:N_NODES]
  deg = deg_raw.reshape(NPAD)[:N_NODES]
  return _tc_dense(x, a, deg, W_self, W_neigh, b.reshape(1, D))


# speedup vs baseline: 1.6732x; 1.0583x over previous
import functools

import jax
import jax.numpy as jnp
from jax import lax
from jax.experimental import pallas as pl
from jax.experimental.pallas import tpu as pltpu
from jax.experimental.pallas import tpu_sc as plsc

N_NODES = 10000
N_EDGES = 160000
D = 256

NC = 2
NS = 16
NW = NC * NS
LANES = 16

PART = 320
NPAD = NW * PART
ACC_ROWS = 336
TRASH = PART
CH = 64
CAP = 128
SCH = 2000
NSCH = N_EDGES // SCH
DEGC = PART // LANES


def _sc_gather_scatter(x, src_ids, dst_ids):
  mesh = plsc.VectorSubcoreMesh(core_axis_name="c", subcore_axis_name="s")

  @functools.partial(
      pl.kernel,
      out_type=[
          jax.ShapeDtypeStruct((NPAD, D), jnp.float32),
          jax.ShapeDtypeStruct((NW, DEGC, LANES), jnp.float32),
      ],
      mesh=mesh,
      compiler_params=pltpu.CompilerParams(needs_layout_passes=False),
      scratch_types=[
          pltpu.VMEM((2 * SCH,), jnp.int32),
          pltpu.VMEM((2 * SCH,), jnp.int32),
          pltpu.VMEM((CAP,), jnp.int32),
          pltpu.VMEM((CAP,), jnp.int32),
          pltpu.VMEM((CH, D), jnp.float32),
          pltpu.VMEM((ACC_ROWS, D), jnp.float32),
          pltpu.VMEM((ACC_ROWS + LANES,), jnp.float32),
          pltpu.VMEM((DEGC, LANES), jnp.float32),
          pltpu.SemaphoreType.DMA,
          pltpu.SemaphoreType.DMA,
      ],
  )
  def k(x_hbm, si_hbm, di_hbm, a_out, deg_out, src_v, dst_v, psrc, pdst,
        gbuf, acc, deg_v, deg2, sem, sem2):
    c = lax.axis_index("c")
    s = lax.axis_index("s")
    wid = s * NC + c
    lo = wid * PART

    zvec = jnp.zeros((LANES,), jnp.float32)
    onevec = jnp.where(lax.iota(jnp.int32, LANES) == 0, 1.0, 0.0)

    def zero_acc(r, _):
      for v in range(D // LANES):
        acc[r, pl.ds(v * LANES, LANES)] = zvec
      return 0
    lax.fori_loop(0, ACC_ROWS, zero_acc, 0)

    def zero_deg(r, _):
      deg_v[pl.ds(r * LANES, LANES)] = zvec
      return 0
    lax.fori_loop(0, (ACC_ROWS + LANES) // LANES, zero_deg, 0)

    def drain():
      pltpu.async_copy(x_hbm.at[psrc.at[pl.ds(0, CH)]], gbuf, sem).wait()

      def group(g, _):
        rows = pdst[pl.ds(g * LANES, LANES)]
        for lane in range(LANES):
          row = rows[lane]
          e = g * LANES + lane
          for v in range(D // LANES):
            sl = pl.ds(v * LANES, LANES)
            acc[row, sl] = acc[row, sl] + gbuf[e, sl]
          dw = deg_v[pl.ds(row, LANES)]
          deg_v[pl.ds(row, LANES)] = dw + onevec
        return 0
      lax.fori_loop(0, CH // LANES, group, 0)

    pltpu.async_copy(si_hbm.at[pl.ds(0, SCH)], src_v.at[pl.ds(0, SCH)], sem2)
    pltpu.async_copy(di_hbm.at[pl.ds(0, SCH)], dst_v.at[pl.ds(0, SCH)], sem2)

    def superstep(g, cursor):
      po = lax.rem(g, 2) * SCH
      pltpu.make_async_copy(si_hbm.at[pl.ds(0, SCH)],
                            src_v.at[pl.ds(0, SCH)], sem2).wait()
      pltpu.make_async_copy(di_hbm.at[pl.ds(0, SCH)],
                            dst_v.at[pl.ds(0, SCH)], sem2).wait()

      @pl.when(g + 1 < NSCH)
      def _prefetch():
        no = SCH - po
        pltpu.async_copy(si_hbm.at[pl.ds((g + 1) * SCH, SCH)],
                         src_v.at[pl.ds(no, SCH)], sem2)
        pltpu.async_copy(di_hbm.at[pl.ds((g + 1) * SCH, SCH)],
                         dst_v.at[pl.ds(no, SCH)], sem2)

      def scan(r, cur):
        dvec = dst_v[pl.ds(po + r * LANES, LANES)]
        mine = (dvec >= lo) & (dvec < lo + PART)
        nmatch = plsc.all_reduce_population_count(mine)[0]

        def slow(cc):
          svec = src_v[pl.ds(po + r * LANES, LANES)]
          mcum = plsc.cumsum(jnp.where(mine, jnp.int32(1), jnp.int32(0)))
          idx = cc + mcum - 1
          plsc.store_scatter(psrc, [idx], svec, mask=mine)
          plsc.store_scatter(pdst, [idx], dvec - lo, mask=mine)
          cc = cc + nmatch

          def do_drain(c2):
            drain()
            psrc[pl.ds(0, LANES)] = psrc[pl.ds(CH, LANES)]
            pdst[pl.ds(0, LANES)] = pdst[pl.ds(CH, LANES)]
            return c2 - CH
          return lax.cond(cc >= CH, do_drain, lambda c2: c2, cc)
        return lax.cond(nmatch > 0, slow, lambda cc: cc, cur)
      return lax.fori_loop(0, SCH // LANES, scan, cursor)
    cursor = lax.fori_loop(0, NSCH, superstep, jnp.int32(0))

    zivec = jnp.zeros((LANES,), jnp.int32)
    tvec = jnp.full((LANES,), TRASH, jnp.int32)
    def pad(w, _):
      off = cursor + w * LANES
      psrc[pl.ds(off, LANES)] = zivec
      pdst[pl.ds(off, LANES)] = tvec
      return 0
    lax.fori_loop(0, CH // LANES, pad, 0)
    drain()

    pltpu.sync_copy(acc.at[pl.ds(0, PART)], a_out.at[pl.ds(lo, PART)])
    def stage_deg(r, _):
      deg2[r, :] = deg_v[pl.ds(r * LANES, LANES)]
      return 0
    lax.fori_loop(0, DEGC, stage_deg, 0)
    pltpu.sync_copy(deg2, deg_out.at[wid])

  return k(x, src_ids, dst_ids)


def _tc_dense(x, a, deg, w_self, w_neigh, b2):
  BLK = 1000
  grid = N_NODES // BLK
  deg3 = deg.reshape(grid, 1, BLK)

  def body(x_ref, a_ref, deg_ref, ws_ref, wn_ref, b_ref, o_ref):
    inv = 1.0 / jnp.maximum(deg_ref[0, 0, :], 1.0)
    acc = jnp.dot(x_ref[...], ws_ref[...], preferred_element_type=jnp.float32)
    agg = jnp.dot(a_ref[...], wn_ref[...], preferred_element_type=jnp.float32)
    o_ref[...] = jnp.maximum(acc + agg * inv[:, None] + b_ref[...], 0.0)

  return pl.pallas_call(
      body,
      grid=(grid,),
      in_specs=[
          pl.BlockSpec((BLK, D), lambda i: (i, 0)),
          pl.BlockSpec((BLK, D), lambda i: (i, 0)),
          pl.BlockSpec((1, 1, BLK), lambda i: (i, 0, 0)),
          pl.BlockSpec((D, D), lambda i: (0, 0)),
          pl.BlockSpec((D, D), lambda i: (0, 0)),
          pl.BlockSpec((1, D), lambda i: (0, 0)),
      ],
      out_specs=pl.BlockSpec((BLK, D), lambda i: (i, 0)),
      out_shape=jax.ShapeDtypeStruct((N_NODES, D), jnp.float32),
  )(x, a, deg3, w_self, w_neigh, b2)


def kernel(x, edge_index, W_self, W_neigh, b):
  a_pad, deg_raw = _sc_gather_scatter(x, edge_index[0], edge_index[1])
  a = a_pad[:N_NODES]
  deg = deg_raw.reshape(NPAD)[:N_NODES]
  return _tc_dense(x, a, deg, W_self, W_neigh, b.reshape(1, D))


# speedup vs baseline: 1.8564x; 1.1095x over previous
import functools

import jax
import jax.numpy as jnp
from jax import lax
from jax.experimental import pallas as pl
from jax.experimental.pallas import tpu as pltpu
from jax.experimental.pallas import tpu_sc as plsc

N_NODES = 10000
N_EDGES = 160000
D = 256

NC = 2
NS = 16
NW = NC * NS
LANES = 16

PART = 320
NPAD = NW * PART
ACC_ROWS = 336
TRASH = PART
CH = 64
CAP = 128
SCH = 2000
NSCH = N_EDGES // SCH
DEGC = PART // LANES


def _sc_gather_scatter(x, src_ids, dst_ids):
  mesh = plsc.VectorSubcoreMesh(core_axis_name="c", subcore_axis_name="s")

  @functools.partial(
      pl.kernel,
      out_type=[
          jax.ShapeDtypeStruct((NPAD, D), jnp.float32),
          jax.ShapeDtypeStruct((NW, DEGC, LANES), jnp.float32),
      ],
      mesh=mesh,
      compiler_params=pltpu.CompilerParams(needs_layout_passes=False),
      scratch_types=[
          pltpu.VMEM((2 * SCH,), jnp.int32),
          pltpu.VMEM((2 * SCH,), jnp.int32),
          pltpu.VMEM((2 * CAP,), jnp.int32),
          pltpu.VMEM((2 * CAP,), jnp.int32),
          pltpu.VMEM((2 * CH, D), jnp.float32),
          pltpu.VMEM((ACC_ROWS, D), jnp.float32),
          pltpu.VMEM((ACC_ROWS + LANES,), jnp.float32),
          pltpu.VMEM((DEGC, LANES), jnp.float32),
          pltpu.SemaphoreType.DMA,
          pltpu.SemaphoreType.DMA,
      ],
  )
  def k(x_hbm, si_hbm, di_hbm, a_out, deg_out, src_v, dst_v, psrc, pdst,
        gbuf, acc, deg_v, deg2, sem, sem2):
    c = lax.axis_index("c")
    s = lax.axis_index("s")
    wid = s * NC + c
    lo = wid * PART

    zvec = jnp.zeros((LANES,), jnp.float32)
    onevec = jnp.where(lax.iota(jnp.int32, LANES) == 0, 1.0, 0.0)

    def zero_acc(r, _):
      for v in range(D // LANES):
        acc[r, pl.ds(v * LANES, LANES)] = zvec
      return 0
    lax.fori_loop(0, ACC_ROWS, zero_acc, 0)

    def zero_deg(r, _):
      deg_v[pl.ds(r * LANES, LANES)] = zvec
      return 0
    lax.fori_loop(0, (ACC_ROWS + LANES) // LANES, zero_deg, 0)

    def fire(pp):
      pltpu.async_copy(x_hbm.at[psrc.at[pl.ds(pp * CAP, CH)]],
                       gbuf.at[pl.ds(pp * CH, CH)], sem)

    def wait_gather(pp):
      pltpu.make_async_copy(x_hbm.at[psrc.at[pl.ds(pp * CAP, CH)]],
                            gbuf.at[pl.ds(pp * CH, CH)], sem).wait()

    def accumulate(pp):
      def group(g, _):
        rows = pdst[pl.ds(pp * CAP + g * LANES, LANES)]
        for lane in range(LANES):
          row = rows[lane]
          e = pp * CH + g * LANES + lane
          for v in range(D // LANES):
            sl = pl.ds(v * LANES, LANES)
            acc[row, sl] = acc[row, sl] + gbuf[e, sl]
          dw = deg_v[pl.ds(row, LANES)]
          deg_v[pl.ds(row, LANES)] = dw + onevec
        return 0
      lax.fori_loop(0, CH // LANES, group, 0)

    pltpu.async_copy(si_hbm.at[pl.ds(0, SCH)], src_v.at[pl.ds(0, SCH)], sem2)
    pltpu.async_copy(di_hbm.at[pl.ds(0, SCH)], dst_v.at[pl.ds(0, SCH)], sem2)

    def superstep(g, state):
      po = lax.rem(g, 2) * SCH
      pltpu.make_async_copy(si_hbm.at[pl.ds(0, SCH)],
                            src_v.at[pl.ds(0, SCH)], sem2).wait()
      pltpu.make_async_copy(di_hbm.at[pl.ds(0, SCH)],
                            dst_v.at[pl.ds(0, SCH)], sem2).wait()

      @pl.when(g + 1 < NSCH)
      def _prefetch():
        no = SCH - po
        pltpu.async_copy(si_hbm.at[pl.ds((g + 1) * SCH, SCH)],
                         src_v.at[pl.ds(no, SCH)], sem2)
        pltpu.async_copy(di_hbm.at[pl.ds((g + 1) * SCH, SCH)],
                         dst_v.at[pl.ds(no, SCH)], sem2)

      def scan(r, st):
        cur, pp, fl = st
        dvec = dst_v[pl.ds(po + r * LANES, LANES)]
        mine = (dvec >= lo) & (dvec < lo + PART)
        nmatch = plsc.all_reduce_population_count(mine)[0]

        def slow(st2):
          cc, p2, f2 = st2
          svec = src_v[pl.ds(po + r * LANES, LANES)]
          mcum = plsc.cumsum(jnp.where(mine, jnp.int32(1), jnp.int32(0)))
          idx = p2 * CAP + cc + mcum - 1
          plsc.store_scatter(psrc, [idx], svec, mask=mine)
          plsc.store_scatter(pdst, [idx], dvec - lo, mask=mine)
          cc = cc + nmatch

          def do_drain(st3):
            c3, p3, f3 = st3

            @pl.when(f3 > 0)
            def _():
              wait_gather(1 - p3)
              accumulate(1 - p3)
            fire(p3)
            p4 = 1 - p3
            psrc[pl.ds(p4 * CAP, LANES)] = psrc[pl.ds(p3 * CAP + CH, LANES)]
            pdst[pl.ds(p4 * CAP, LANES)] = pdst[pl.ds(p3 * CAP + CH, LANES)]
            return (c3 - CH, p4, jnp.int32(1))
          return lax.cond(cc >= CH, do_drain, lambda s3: s3, (cc, p2, f2))
        return lax.cond(nmatch > 0, slow, lambda s2: s2, (cur, pp, fl))
      return lax.fori_loop(0, SCH // LANES, scan, state)
    state = lax.fori_loop(0, NSCH, superstep,
                          (jnp.int32(0), jnp.int32(0), jnp.int32(0)))
    cursor, pp, fl = state

    @pl.when(fl > 0)
    def _():
      wait_gather(1 - pp)
      accumulate(1 - pp)

    zivec = jnp.zeros((LANES,), jnp.int32)
    tvec = jnp.full((LANES,), TRASH, jnp.int32)
    def pad(w, _):
      off = pp * CAP + cursor + w * LANES
      psrc[pl.ds(off, LANES)] = zivec
      pdst[pl.ds(off, LANES)] = tvec
      return 0
    lax.fori_loop(0, CH // LANES, pad, 0)
    fire(pp)
    wait_gather(pp)
    accumulate(pp)

    pltpu.sync_copy(acc.at[pl.ds(0, PART)], a_out.at[pl.ds(lo, PART)])
    def stage_deg(r, _):
      deg2[r, :] = deg_v[pl.ds(r * LANES, LANES)]
      return 0
    lax.fori_loop(0, DEGC, stage_deg, 0)
    pltpu.sync_copy(deg2, deg_out.at[wid])

  return k(x, src_ids, dst_ids)


def _tc_dense(x, a, deg, w_self, w_neigh, b2):
  BLK = 1000
  grid = N_NODES // BLK
  deg3 = deg.reshape(grid, 1, BLK)

  def body(x_ref, a_ref, deg_ref, ws_ref, wn_ref, b_ref, o_ref):
    inv = 1.0 / jnp.maximum(deg_ref[0, 0, :], 1.0)
    acc = jnp.dot(x_ref[...], ws_ref[...], preferred_element_type=jnp.float32)
    agg = jnp.dot(a_ref[...], wn_ref[...], preferred_element_type=jnp.float32)
    o_ref[...] = jnp.maximum(acc + agg * inv[:, None] + b_ref[...], 0.0)

  return pl.pallas_call(
      body,
      grid=(grid,),
      in_specs=[
          pl.BlockSpec((BLK, D), lambda i: (i, 0)),
          pl.BlockSpec((BLK, D), lambda i: (i, 0)),
          pl.BlockSpec((1, 1, BLK), lambda i: (i, 0, 0)),
          pl.BlockSpec((D, D), lambda i: (0, 0)),
          pl.BlockSpec((D, D), lambda i: (0, 0)),
          pl.BlockSpec((1, D), lambda i: (0, 0)),
      ],
      out_specs=pl.BlockSpec((BLK, D), lambda i: (i, 0)),
      out_shape=jax.ShapeDtypeStruct((N_NODES, D), jnp.float32),
  )(x, a, deg3, w_self, w_neigh, b2)


def kernel(x, edge_index, W_self, W_neigh, b):
  a_pad, deg_raw = _sc_gather_scatter(x, edge_index[0], edge_index[1])
  a = a_pad[:N_NODES]
  deg = deg_raw.reshape(NPAD)[:N_NODES]
  return _tc_dense(x, a, deg, W_self, W_neigh, b.reshape(1, D))


# speedup vs baseline: 2.1067x; 1.1349x over previous
import functools

import jax
import jax.numpy as jnp
from jax import lax
from jax.experimental import pallas as pl
from jax.experimental.pallas import tpu as pltpu
from jax.experimental.pallas import tpu_sc as plsc

N_NODES = 10000
N_EDGES = 160000
D = 256

NC = 2
NS = 16
NW = NC * NS
LANES = 16

PART = 320
NPAD = NW * PART
ACC_ROWS = 336
TRASH = PART
CH = 64
CAP = 128
SCH = 2000
NSCH = N_EDGES // SCH
DEGC = PART // LANES


def _sc_gather_scatter(x, src_ids, dst_ids):
  mesh = plsc.VectorSubcoreMesh(core_axis_name="c", subcore_axis_name="s")

  @functools.partial(
      pl.kernel,
      out_type=[
          jax.ShapeDtypeStruct((NPAD, D), jnp.float32),
          jax.ShapeDtypeStruct((NW, DEGC, LANES), jnp.float32),
      ],
      mesh=mesh,
      compiler_params=pltpu.CompilerParams(needs_layout_passes=False),
      scratch_types=[
          pltpu.VMEM((2 * SCH,), jnp.int32),
          pltpu.VMEM((2 * SCH,), jnp.int32),
          pltpu.VMEM((2 * CAP,), jnp.int32),
          pltpu.VMEM((2 * CAP,), jnp.int32),
          pltpu.VMEM((2 * CH, D), jnp.float32),
          pltpu.VMEM((ACC_ROWS, D), jnp.float32),
          pltpu.VMEM((ACC_ROWS + LANES,), jnp.float32),
          pltpu.VMEM((DEGC, LANES), jnp.float32),
          pltpu.SemaphoreType.DMA,
          pltpu.SemaphoreType.DMA,
      ],
  )
  def k(x_hbm, si_hbm, di_hbm, a_out, deg_out, src_v, dst_v, psrc, pdst,
        gbuf, acc, deg_v, deg2, sem, sem2):
    c = lax.axis_index("c")
    s = lax.axis_index("s")
    wid = s * NC + c
    lo = wid * PART

    zvec = jnp.zeros((LANES,), jnp.float32)
    onevec = jnp.where(lax.iota(jnp.int32, LANES) == 0, 1.0, 0.0)

    def zero_acc(r, _):
      for v in range(D // LANES):
        acc[r, pl.ds(v * LANES, LANES)] = zvec
      return 0
    lax.fori_loop(0, ACC_ROWS, zero_acc, 0)

    def zero_deg(r, _):
      deg_v[pl.ds(r * LANES, LANES)] = zvec
      return 0
    lax.fori_loop(0, (ACC_ROWS + LANES) // LANES, zero_deg, 0)

    def fire(pp):
      pltpu.async_copy(x_hbm.at[psrc.at[pl.ds(pp * CAP, CH)]],
                       gbuf.at[pl.ds(pp * CH, CH)], sem)

    def wait_gather(pp):
      pltpu.make_async_copy(x_hbm.at[psrc.at[pl.ds(pp * CAP, CH)]],
                            gbuf.at[pl.ds(pp * CH, CH)], sem).wait()

    def accumulate(pp):
      def group(g, _):
        rows = pdst[pl.ds(pp * CAP + g * LANES, LANES)]
        for lane in range(LANES):
          row = rows[lane]
          e = pp * CH + g * LANES + lane
          for v in range(D // LANES):
            sl = pl.ds(v * LANES, LANES)
            plsc.addupdate(acc.at[row, sl], gbuf[e, sl])
          dw = deg_v[pl.ds(row, LANES)]
          deg_v[pl.ds(row, LANES)] = dw + onevec
        return 0
      lax.fori_loop(0, CH // LANES, group, 0)

    pltpu.async_copy(si_hbm.at[pl.ds(0, SCH)], src_v.at[pl.ds(0, SCH)], sem2)
    pltpu.async_copy(di_hbm.at[pl.ds(0, SCH)], dst_v.at[pl.ds(0, SCH)], sem2)

    def superstep(g, state):
      po = lax.rem(g, 2) * SCH
      pltpu.make_async_copy(si_hbm.at[pl.ds(0, SCH)],
                            src_v.at[pl.ds(0, SCH)], sem2).wait()
      pltpu.make_async_copy(di_hbm.at[pl.ds(0, SCH)],
                            dst_v.at[pl.ds(0, SCH)], sem2).wait()

      @pl.when(g + 1 < NSCH)
      def _prefetch():
        no = SCH - po
        pltpu.async_copy(si_hbm.at[pl.ds((g + 1) * SCH, SCH)],
                         src_v.at[pl.ds(no, SCH)], sem2)
        pltpu.async_copy(di_hbm.at[pl.ds((g + 1) * SCH, SCH)],
                         dst_v.at[pl.ds(no, SCH)], sem2)

      def scan(r, st):
        cur, pp, fl = st
        dvec = dst_v[pl.ds(po + r * LANES, LANES)]
        mine = (dvec >= lo) & (dvec < lo + PART)
        nmatch = plsc.all_reduce_population_count(mine)[0]

        def slow(st2):
          cc, p2, f2 = st2
          svec = src_v[pl.ds(po + r * LANES, LANES)]
          mcum = plsc.cumsum(jnp.where(mine, jnp.int32(1), jnp.int32(0)))
          idx = p2 * CAP + cc + mcum - 1
          plsc.store_scatter(psrc, [idx], svec, mask=mine)
          plsc.store_scatter(pdst, [idx], dvec - lo, mask=mine)
          cc = cc + nmatch

          def do_drain(st3):
            c3, p3, f3 = st3

            @pl.when(f3 > 0)
            def _():
              wait_gather(1 - p3)
              accumulate(1 - p3)
            fire(p3)
            p4 = 1 - p3
            psrc[pl.ds(p4 * CAP, LANES)] = psrc[pl.ds(p3 * CAP + CH, LANES)]
            pdst[pl.ds(p4 * CAP, LANES)] = pdst[pl.ds(p3 * CAP + CH, LANES)]
            return (c3 - CH, p4, jnp.int32(1))
          return lax.cond(cc >= CH, do_drain, lambda s3: s3, (cc, p2, f2))
        return lax.cond(nmatch > 0, slow, lambda s2: s2, (cur, pp, fl))
      return lax.fori_loop(0, SCH // LANES, scan, state)
    state = lax.fori_loop(0, NSCH, superstep,
                          (jnp.int32(0), jnp.int32(0), jnp.int32(0)))
    cursor, pp, fl = state

    @pl.when(fl > 0)
    def _():
      wait_gather(1 - pp)
      accumulate(1 - pp)

    zivec = jnp.zeros((LANES,), jnp.int32)
    tvec = jnp.full((LANES,), TRASH, jnp.int32)
    def pad(w, _):
      off = pp * CAP + cursor + w * LANES
      psrc[pl.ds(off, LANES)] = zivec
      pdst[pl.ds(off, LANES)] = tvec
      return 0
    lax.fori_loop(0, CH // LANES, pad, 0)
    fire(pp)
    wait_gather(pp)
    accumulate(pp)

    pltpu.sync_copy(acc.at[pl.ds(0, PART)], a_out.at[pl.ds(lo, PART)])
    def stage_deg(r, _):
      deg2[r, :] = deg_v[pl.ds(r * LANES, LANES)]
      return 0
    lax.fori_loop(0, DEGC, stage_deg, 0)
    pltpu.sync_copy(deg2, deg_out.at[wid])

  return k(x, src_ids, dst_ids)


def _tc_dense(x, a, deg, w_self, w_neigh, b2):
  BLK = 1000
  grid = N_NODES // BLK
  deg3 = deg.reshape(grid, 1, BLK)

  def body(x_ref, a_ref, deg_ref, ws_ref, wn_ref, b_ref, o_ref):
    inv = 1.0 / jnp.maximum(deg_ref[0, 0, :], 1.0)
    acc = jnp.dot(x_ref[...], ws_ref[...], preferred_element_type=jnp.float32)
    agg = jnp.dot(a_ref[...], wn_ref[...], preferred_element_type=jnp.float32)
    o_ref[...] = jnp.maximum(acc + agg * inv[:, None] + b_ref[...], 0.0)

  return pl.pallas_call(
      body,
      grid=(grid,),
      in_specs=[
          pl.BlockSpec((BLK, D), lambda i: (i, 0)),
          pl.BlockSpec((BLK, D), lambda i: (i, 0)),
          pl.BlockSpec((1, 1, BLK), lambda i: (i, 0, 0)),
          pl.BlockSpec((D, D), lambda i: (0, 0)),
          pl.BlockSpec((D, D), lambda i: (0, 0)),
          pl.BlockSpec((1, D), lambda i: (0, 0)),
      ],
      out_specs=pl.BlockSpec((BLK, D), lambda i: (i, 0)),
      out_shape=jax.ShapeDtypeStruct((N_NODES, D), jnp.float32),
  )(x, a, deg3, w_self, w_neigh, b2)


def kernel(x, edge_index, W_self, W_neigh, b):
  a_pad, deg_raw = _sc_gather_scatter(x, edge_index[0], edge_index[1])
  a = a_pad[:N_NODES]
  deg = deg_raw.reshape(NPAD)[:N_NODES]
  return _tc_dense(x, a, deg, W_self, W_neigh, b.reshape(1, D))


# speedup vs baseline: 2.6321x; 1.2494x over previous
import functools

import jax
import jax.numpy as jnp
from jax import lax
from jax.experimental import pallas as pl
from jax.experimental.pallas import tpu as pltpu
from jax.experimental.pallas import tpu_sc as plsc

N_NODES = 10000
N_EDGES = 160000
D = 256

NC = 2
NS = 16
NW = NC * NS
LANES = 16

PART = 320
NPAD = NW * PART
ACC_ROWS = 336
TRASH = PART
CH = 64
CAP = 144
SCH = 1600
NSCH = N_EDGES // SCH
BATCH = 4
DEGC = PART // LANES


def _sc_gather_scatter(x, src_ids, dst_ids):
  mesh = plsc.VectorSubcoreMesh(core_axis_name="c", subcore_axis_name="s")

  @functools.partial(
      pl.kernel,
      out_type=[
          jax.ShapeDtypeStruct((NPAD, D), jnp.float32),
          jax.ShapeDtypeStruct((NW, DEGC, LANES), jnp.float32),
      ],
      mesh=mesh,
      compiler_params=pltpu.CompilerParams(needs_layout_passes=False),
      scratch_types=[
          pltpu.VMEM((2 * SCH,), jnp.int32),
          pltpu.VMEM((2 * SCH,), jnp.int32),
          pltpu.VMEM((2 * CAP,), jnp.int32),
          pltpu.VMEM((2 * CAP,), jnp.int32),
          pltpu.VMEM((2 * CH, D), jnp.float32),
          pltpu.VMEM((ACC_ROWS, D), jnp.float32),
          pltpu.VMEM((ACC_ROWS + LANES,), jnp.float32),
          pltpu.VMEM((DEGC, LANES), jnp.float32),
          pltpu.SemaphoreType.DMA,
          pltpu.SemaphoreType.DMA,
      ],
  )
  def k(x_hbm, si_hbm, di_hbm, a_out, deg_out, src_v, dst_v, psrc, pdst,
        gbuf, acc, deg_v, deg2, sem, sem2):
    c = lax.axis_index("c")
    s = lax.axis_index("s")
    wid = s * NC + c
    lo = wid * PART

    zvec = jnp.zeros((LANES,), jnp.float32)
    onevec = jnp.where(lax.iota(jnp.int32, LANES) == 0, 1.0, 0.0)

    def zero_acc(r, _):
      for v in range(D // LANES):
        acc[r, pl.ds(v * LANES, LANES)] = zvec
      return 0
    lax.fori_loop(0, ACC_ROWS, zero_acc, 0)

    def zero_deg(r, _):
      deg_v[pl.ds(r * LANES, LANES)] = zvec
      return 0
    lax.fori_loop(0, (ACC_ROWS + LANES) // LANES, zero_deg, 0)

    def fire(pp):
      pltpu.async_copy(x_hbm.at[psrc.at[pl.ds(pp * CAP, CH)]],
                       gbuf.at[pl.ds(pp * CH, CH)], sem)

    def wait_gather(pp):
      pltpu.make_async_copy(x_hbm.at[psrc.at[pl.ds(pp * CAP, CH)]],
                            gbuf.at[pl.ds(pp * CH, CH)], sem).wait()

    def accumulate(pp):
      def group(g, _):
        rows = pdst[pl.ds(pp * CAP + g * LANES, LANES)]
        for lane in range(LANES):
          row = rows[lane]
          e = pp * CH + g * LANES + lane
          for v in range(D // LANES):
            sl = pl.ds(v * LANES, LANES)
            plsc.addupdate(acc.at[row, sl], gbuf[e, sl])
          dw = deg_v[pl.ds(row, LANES)]
          deg_v[pl.ds(row, LANES)] = dw + onevec
        return 0
      lax.fori_loop(0, CH // LANES, group, 0)

    pltpu.async_copy(si_hbm.at[pl.ds(0, SCH)], src_v.at[pl.ds(0, SCH)], sem2)
    pltpu.async_copy(di_hbm.at[pl.ds(0, SCH)], dst_v.at[pl.ds(0, SCH)], sem2)

    def superstep(g, state):
      po = lax.rem(g, 2) * SCH
      pltpu.make_async_copy(si_hbm.at[pl.ds(0, SCH)],
                            src_v.at[pl.ds(0, SCH)], sem2).wait()
      pltpu.make_async_copy(di_hbm.at[pl.ds(0, SCH)],
                            dst_v.at[pl.ds(0, SCH)], sem2).wait()

      @pl.when(g + 1 < NSCH)
      def _prefetch():
        no = SCH - po
        pltpu.async_copy(si_hbm.at[pl.ds((g + 1) * SCH, SCH)],
                         src_v.at[pl.ds(no, SCH)], sem2)
        pltpu.async_copy(di_hbm.at[pl.ds((g + 1) * SCH, SCH)],
                         dst_v.at[pl.ds(no, SCH)], sem2)

      def scan(r, st):
        cur, pp, fl = st
        cc = cur
        for u in range(BATCH):
          off = po + (r * BATCH + u) * LANES
          dvec = dst_v[pl.ds(off, LANES)]
          svec = src_v[pl.ds(off, LANES)]
          mine = (dvec >= lo) & (dvec < lo + PART)
          mcum = plsc.cumsum(jnp.where(mine, jnp.int32(1), jnp.int32(0)))
          idx = pp * CAP + cc + mcum - 1
          plsc.store_scatter(psrc, [idx], svec, mask=mine)
          plsc.store_scatter(pdst, [idx], dvec - lo, mask=mine)
          cc = cc + mcum[LANES - 1]

        def do_drain(st3):
          c3, p3, f3 = st3

          @pl.when(f3 > 0)
          def _():
            wait_gather(1 - p3)
            accumulate(1 - p3)
          fire(p3)
          p4 = 1 - p3
          for w in range(BATCH):
            psrc[pl.ds(p4 * CAP + w * LANES, LANES)] = (
                psrc[pl.ds(p3 * CAP + CH + w * LANES, LANES)])
            pdst[pl.ds(p4 * CAP + w * LANES, LANES)] = (
                pdst[pl.ds(p3 * CAP + CH + w * LANES, LANES)])
          return (c3 - CH, p4, jnp.int32(1))
        return lax.cond(cc >= CH, do_drain, lambda s3: s3, (cc, pp, fl))
      return lax.fori_loop(0, SCH // (LANES * BATCH), scan, state)
    state = lax.fori_loop(0, NSCH, superstep,
                          (jnp.int32(0), jnp.int32(0), jnp.int32(0)))
    cursor, pp, fl = state

    @pl.when(fl > 0)
    def _():
      wait_gather(1 - pp)
      accumulate(1 - pp)

    zivec = jnp.zeros((LANES,), jnp.int32)
    tvec = jnp.full((LANES,), TRASH, jnp.int32)
    def pad(w, _):
      off = pp * CAP + cursor + w * LANES
      psrc[pl.ds(off, LANES)] = zivec
      pdst[pl.ds(off, LANES)] = tvec
      return 0
    lax.fori_loop(0, CH // LANES, pad, 0)
    fire(pp)
    wait_gather(pp)
    accumulate(pp)

    pltpu.sync_copy(acc.at[pl.ds(0, PART)], a_out.at[pl.ds(lo, PART)])
    def stage_deg(r, _):
      deg2[r, :] = deg_v[pl.ds(r * LANES, LANES)]
      return 0
    lax.fori_loop(0, DEGC, stage_deg, 0)
    pltpu.sync_copy(deg2, deg_out.at[wid])

  return k(x, src_ids, dst_ids)


def _tc_dense(x, a, deg, w_self, w_neigh, b2):
  BLK = 1000
  grid = N_NODES // BLK
  deg3 = deg.reshape(grid, 1, BLK)

  def body(x_ref, a_ref, deg_ref, ws_ref, wn_ref, b_ref, o_ref):
    inv = 1.0 / jnp.maximum(deg_ref[0, 0, :], 1.0)
    acc = jnp.dot(x_ref[...], ws_ref[...], preferred_element_type=jnp.float32)
    agg = jnp.dot(a_ref[...], wn_ref[...], preferred_element_type=jnp.float32)
    o_ref[...] = jnp.maximum(acc + agg * inv[:, None] + b_ref[...], 0.0)

  return pl.pallas_call(
      body,
      grid=(grid,),
      in_specs=[
          pl.BlockSpec((BLK, D), lambda i: (i, 0)),
          pl.BlockSpec((BLK, D), lambda i: (i, 0)),
          pl.BlockSpec((1, 1, BLK), lambda i: (i, 0, 0)),
          pl.BlockSpec((D, D), lambda i: (0, 0)),
          pl.BlockSpec((D, D), lambda i: (0, 0)),
          pl.BlockSpec((1, D), lambda i: (0, 0)),
      ],
      out_specs=pl.BlockSpec((BLK, D), lambda i: (i, 0)),
      out_shape=jax.ShapeDtypeStruct((N_NODES, D), jnp.float32),
  )(x, a, deg3, w_self, w_neigh, b2)


def kernel(x, edge_index, W_self, W_neigh, b):
  a_pad, deg_raw = _sc_gather_scatter(x, edge_index[0], edge_index[1])
  a = a_pad[:N_NODES]
  deg = deg_raw.reshape(NPAD)[:N_NODES]
  return _tc_dense(x, a, deg, W_self, W_neigh, b.reshape(1, D))
